# initial kernel scaffold (unmeasured)
import jax
import jax.numpy as jnp
from jax import lax
from jax.experimental import pallas as pl
from jax.experimental.pallas import tpu as pltpu

N_DEV = 8
M = 4096
N = 2048
CH = M // N_DEV


def kernel(x, w_mat):
    partial = jnp.dot(x, w_mat, preferred_element_type=jnp.float32)
    partial = partial.astype(jnp.bfloat16)

    def body(p_ref, out_ref, recv_ref, send_ref, rs_ssem, rs_rsem,
             ag_ssem, ag_rsem):
        i = lax.axis_index("i")
        left = lax.rem(i + N_DEV - 1, N_DEV)
        right = lax.rem(i + 1, N_DEV)

        barrier = pltpu.get_barrier_semaphore()
        for nbr in (left, right):
            pl.semaphore_signal(barrier, inc=1, device_id=(nbr,),
                                device_id_type=pl.DeviceIdType.MESH)
        pl.semaphore_wait(barrier, 2)

        send_ref[...] = p_ref[pl.ds(i * CH, CH), :]
        for s in range(N_DEV - 1):
            rdma = pltpu.make_async_remote_copy(
                src_ref=send_ref,
                dst_ref=recv_ref.at[s],
                send_sem=rs_ssem.at[s],
                recv_sem=rs_rsem.at[s],
                device_id=(right,),
                device_id_type=pl.DeviceIdType.MESH,
            )
            rdma.start()
            rdma.wait()
            c = lax.rem(i + 2 * N_DEV - s - 1, N_DEV)
            acc = (p_ref[pl.ds(c * CH, CH), :].astype(jnp.float32)
                   + recv_ref[s].astype(jnp.float32))
            if s < N_DEV - 2:
                send_ref[...] = acc.astype(jnp.bfloat16)
            else:
                out_ref[pl.ds(c * CH, CH), :] = (
                    jnp.maximum(acc, 0.0).astype(jnp.bfloat16))

        for s in range(N_DEV - 1):
            c = lax.rem(i + 1 + 2 * N_DEV - s, N_DEV)
            rows = pl.ds(c * CH, CH)
            rdma = pltpu.make_async_remote_copy(
                src_ref=out_ref.at[rows, :],
                dst_ref=out_ref.at[rows, :],
                send_sem=ag_ssem.at[s],
                recv_sem=ag_rsem.at[s],
                device_id=(right,),
                device_id_type=pl.DeviceIdType.MESH,
            )
            rdma.start()
            rdma.wait()

        amax = jnp.float32(0.0)
        for c in range(N_DEV):
            chunk = out_ref[c * CH:(c + 1) * CH, :].astype(jnp.float32)
            amax = jnp.maximum(amax, jnp.max(chunk))
        scale = amax / 448.0
        for c in range(N_DEV):
            chunk = out_ref[c * CH:(c + 1) * CH, :].astype(jnp.float32)
            q = (chunk / scale).astype(jnp.float8_e4m3fn)
            out_ref[c * CH:(c + 1) * CH, :] = (
                (q.astype(jnp.float32) * scale).astype(jnp.bfloat16))

    return pl.pallas_call(
        body,
        out_shape=jax.ShapeDtypeStruct((M, N), jnp.bfloat16),
        in_specs=[pl.BlockSpec(memory_space=pltpu.VMEM)],
        out_specs=pl.BlockSpec(memory_space=pltpu.VMEM),
        scratch_shapes=[
            pltpu.VMEM((N_DEV - 1, CH, N), jnp.bfloat16),
            pltpu.VMEM((CH, N), jnp.bfloat16),
            pltpu.SemaphoreType.DMA((N_DEV - 1,)),
            pltpu.SemaphoreType.DMA((N_DEV - 1,)),
            pltpu.SemaphoreType.DMA((N_DEV - 1,)),
            pltpu.SemaphoreType.DMA((N_DEV - 1,)),
        ],
        compiler_params=pltpu.CompilerParams(collective_id=0),
    )(partial)


# baseline (device time: 402439 ns/iter reference)
import jax
import jax.numpy as jnp
from jax import lax
from jax.experimental import pallas as pl
from jax.experimental.pallas import tpu as pltpu

N_DEV = 8
M = 4096
N = 2048
CH = M // N_DEV


def kernel(x, w_mat):
    partial = jnp.dot(x, w_mat, preferred_element_type=jnp.float32)
    partial = partial.astype(jnp.bfloat16)

    def body(p_ref, out_ref, recv_ref, send_ref, rs_ssem, rs_rsem,
             ag_ssem, ag_rsem):
        i = lax.axis_index("i")
        left = lax.rem(i + N_DEV - 1, N_DEV)
        right = lax.rem(i + 1, N_DEV)

        barrier = pltpu.get_barrier_semaphore()
        for nbr in (left, right):
            pl.semaphore_signal(barrier, inc=1, device_id=(nbr,),
                                device_id_type=pl.DeviceIdType.MESH)
        pl.semaphore_wait(barrier, 2)

        send_ref[...] = p_ref[pl.ds(i * CH, CH), :]
        for s in range(N_DEV - 1):
            rdma = pltpu.make_async_remote_copy(
                src_ref=send_ref,
                dst_ref=recv_ref.at[s],
                send_sem=rs_ssem.at[s],
                recv_sem=rs_rsem.at[s],
                device_id=(right,),
                device_id_type=pl.DeviceIdType.MESH,
            )
            rdma.start()
            rdma.wait()
            c = lax.rem(i + 2 * N_DEV - s - 1, N_DEV)
            acc = (p_ref[pl.ds(c * CH, CH), :].astype(jnp.float32)
                   + recv_ref[s].astype(jnp.float32))
            if s < N_DEV - 2:
                send_ref[...] = acc.astype(jnp.bfloat16)
            else:
                out_ref[pl.ds(c * CH, CH), :] = (
                    jnp.maximum(acc, 0.0).astype(jnp.bfloat16))

        for s in range(N_DEV - 1):
            c = lax.rem(i + 1 + 2 * N_DEV - s, N_DEV)
            rows = pl.ds(c * CH, CH)
            rdma = pltpu.make_async_remote_copy(
                src_ref=out_ref.at[rows, :],
                dst_ref=out_ref.at[rows, :],
                send_sem=ag_ssem.at[s],
                recv_sem=ag_rsem.at[s],
                device_id=(right,),
                device_id_type=pl.DeviceIdType.MESH,
            )
            rdma.start()
            rdma.wait()

        amax = jnp.float32(0.0)
        for c in range(N_DEV):
            chunk = out_ref[c * CH:(c + 1) * CH, :].astype(jnp.float32)
            amax = jnp.maximum(amax, jnp.max(chunk))
        scale = amax / 448.0
        for c in range(N_DEV):
            chunk = out_ref[c * CH:(c + 1) * CH, :].astype(jnp.float32)
            q = (chunk / scale).astype(jnp.float8_e4m3fn)
            out_ref[c * CH:(c + 1) * CH, :] = (
                (q.astype(jnp.float32) * scale).astype(jnp.bfloat16))

    return pl.pallas_call(
        body,
        out_shape=jax.ShapeDtypeStruct((M, N), jnp.bfloat16),
        in_specs=[pl.BlockSpec(memory_space=pltpu.VMEM)],
        out_specs=pl.BlockSpec(memory_space=pltpu.VMEM),
        scratch_shapes=[
            pltpu.VMEM((N_DEV - 1, CH, N), jnp.bfloat16),
            pltpu.VMEM((CH, N), jnp.bfloat16),
            pltpu.SemaphoreType.DMA((N_DEV - 1,)),
            pltpu.SemaphoreType.DMA((N_DEV - 1,)),
            pltpu.SemaphoreType.DMA((N_DEV - 1,)),
            pltpu.SemaphoreType.DMA((N_DEV - 1,)),
        ],
        compiler_params=pltpu.CompilerParams(
            collective_id=0,
            vmem_limit_bytes=64 * 1024 * 1024,
        ),
    )(partial)


# device time: 250922 ns/iter; 1.6038x vs baseline; 1.6038x over previous
import jax
import jax.numpy as jnp
from jax import lax
from jax.experimental import pallas as pl
from jax.experimental.pallas import tpu as pltpu

N_DEV = 8
M = 4096
N = 2048
NH = N // 2
CH = M // N_DEV


def kernel(x, w_mat):
    partial = jnp.dot(x, w_mat, preferred_element_type=jnp.float32)
    partial = partial.astype(jnp.bfloat16)

    def body(p_ref, out_ref, recv_a, recv_b, send_a, send_b,
             rs_ssem_a, rs_rsem_a, rs_ssem_b, rs_rsem_b,
             ag_ssem_a, ag_rsem_a, ag_ssem_b, ag_rsem_b):
        i = lax.axis_index("i")
        left = lax.rem(i + N_DEV - 1, N_DEV)
        right = lax.rem(i + 1, N_DEV)

        barrier = pltpu.get_barrier_semaphore()
        for nbr in (left, right):
            pl.semaphore_signal(barrier, inc=1, device_id=(nbr,),
                                device_id_type=pl.DeviceIdType.MESH)
        pl.semaphore_wait(barrier, 2)

        send_a[...] = p_ref[pl.ds(i * CH, CH), 0:NH]
        send_b[...] = p_ref[pl.ds(i * CH, CH), NH:N]
        for s in range(N_DEV - 1):
            rdma_a = pltpu.make_async_remote_copy(
                src_ref=send_a, dst_ref=recv_a.at[s],
                send_sem=rs_ssem_a.at[s], recv_sem=rs_rsem_a.at[s],
                device_id=(right,), device_id_type=pl.DeviceIdType.MESH)
            rdma_b = pltpu.make_async_remote_copy(
                src_ref=send_b, dst_ref=recv_b.at[s],
                send_sem=rs_ssem_b.at[s], recv_sem=rs_rsem_b.at[s],
                device_id=(left,), device_id_type=pl.DeviceIdType.MESH)
            rdma_a.start()
            rdma_b.start()
            rdma_a.wait()
            rdma_b.wait()
            ca = lax.rem(i + 2 * N_DEV - s - 1, N_DEV)
            cb = lax.rem(i + s + 1, N_DEV)
            acc_a = (p_ref[pl.ds(ca * CH, CH), 0:NH].astype(jnp.float32)
                     + recv_a[s].astype(jnp.float32))
            acc_b = (p_ref[pl.ds(cb * CH, CH), NH:N].astype(jnp.float32)
                     + recv_b[s].astype(jnp.float32))
            if s < N_DEV - 2:
                send_a[...] = acc_a.astype(jnp.bfloat16)
                send_b[...] = acc_b.astype(jnp.bfloat16)
            else:
                out_ref[pl.ds(ca * CH, CH), 0:NH] = (
                    jnp.maximum(acc_a, 0.0).astype(jnp.bfloat16))
                out_ref[pl.ds(cb * CH, CH), NH:N] = (
                    jnp.maximum(acc_b, 0.0).astype(jnp.bfloat16))

        for s in range(N_DEV - 1):
            ca = lax.rem(i + 1 + 2 * N_DEV - s, N_DEV)
            cb = lax.rem(i + N_DEV - 1 + s, N_DEV)
            rows_a = pl.ds(ca * CH, CH)
            rows_b = pl.ds(cb * CH, CH)
            rdma_a = pltpu.make_async_remote_copy(
                src_ref=out_ref.at[rows_a, 0:NH],
                dst_ref=out_ref.at[rows_a, 0:NH],
                send_sem=ag_ssem_a.at[s], recv_sem=ag_rsem_a.at[s],
                device_id=(right,), device_id_type=pl.DeviceIdType.MESH)
            rdma_b = pltpu.make_async_remote_copy(
                src_ref=out_ref.at[rows_b, NH:N],
                dst_ref=out_ref.at[rows_b, NH:N],
                send_sem=ag_ssem_b.at[s], recv_sem=ag_rsem_b.at[s],
                device_id=(left,), device_id_type=pl.DeviceIdType.MESH)
            rdma_a.start()
            rdma_b.start()
            rdma_a.wait()
            rdma_b.wait()

        amax = jnp.float32(0.0)
        for c in range(N_DEV):
            chunk = out_ref[c * CH:(c + 1) * CH, :].astype(jnp.float32)
            amax = jnp.maximum(amax, jnp.max(chunk))
        scale = amax / 448.0
        for c in range(N_DEV):
            chunk = out_ref[c * CH:(c + 1) * CH, :].astype(jnp.float32)
            q = (chunk / scale).astype(jnp.float8_e4m3fn)
            out_ref[c * CH:(c + 1) * CH, :] = (
                (q.astype(jnp.float32) * scale).astype(jnp.bfloat16))

    nsem = N_DEV - 1
    return pl.pallas_call(
        body,
        out_shape=jax.ShapeDtypeStruct((M, N), jnp.bfloat16),
        in_specs=[pl.BlockSpec(memory_space=pltpu.VMEM)],
        out_specs=pl.BlockSpec(memory_space=pltpu.VMEM),
        scratch_shapes=[
            pltpu.VMEM((nsem, CH, NH), jnp.bfloat16),
            pltpu.VMEM((nsem, CH, NH), jnp.bfloat16),
            pltpu.VMEM((CH, NH), jnp.bfloat16),
            pltpu.VMEM((CH, NH), jnp.bfloat16),
            pltpu.SemaphoreType.DMA((nsem,)),
            pltpu.SemaphoreType.DMA((nsem,)),
            pltpu.SemaphoreType.DMA((nsem,)),
            pltpu.SemaphoreType.DMA((nsem,)),
            pltpu.SemaphoreType.DMA((nsem,)),
            pltpu.SemaphoreType.DMA((nsem,)),
            pltpu.SemaphoreType.DMA((nsem,)),
            pltpu.SemaphoreType.DMA((nsem,)),
        ],
        compiler_params=pltpu.CompilerParams(
            collective_id=0,
            vmem_limit_bytes=64 * 1024 * 1024,
        ),
    )(partial)


# device time: 199965 ns/iter; 2.0125x vs baseline; 1.2548x over previous
import jax
import jax.numpy as jnp
from jax import lax
from jax.experimental import pallas as pl
from jax.experimental.pallas import tpu as pltpu

N_DEV = 8
M = 4096
N = 2048
NH = N // 2
CH = M // N_DEV


def kernel(x, w_mat):
    partial = jnp.dot(x, w_mat, preferred_element_type=jnp.float32)
    partial = partial.astype(jnp.bfloat16)

    def body(p_ref, out_ref, recv_a, recv_b, send_a, send_b,
             acc_a, acc_b, ag_a, ag_b, amax_buf, stage,
             rs_ssem_a, rs_rsem_a, rs_ssem_b, rs_rsem_b,
             ag_ssem_a, ag_rsem_a, ag_ssem_b, ag_rsem_b,
             am_ssem, am_rsem, copy_sem):
        i = lax.axis_index("i")
        left = lax.rem(i + N_DEV - 1, N_DEV)
        right = lax.rem(i + 1, N_DEV)

        barrier = pltpu.get_barrier_semaphore()
        for nbr in (left, right):
            pl.semaphore_signal(barrier, inc=1, device_id=(nbr,),
                                device_id_type=pl.DeviceIdType.MESH)
        pl.semaphore_wait(barrier, 2)

        send_a[...] = p_ref[pl.ds(i * CH, CH), 0:NH]
        send_b[...] = p_ref[pl.ds(i * CH, CH), NH:N]
        for s in range(N_DEV - 1):
            rdma_a = pltpu.make_async_remote_copy(
                src_ref=send_a, dst_ref=recv_a.at[s],
                send_sem=rs_ssem_a.at[s], recv_sem=rs_rsem_a.at[s],
                device_id=(right,), device_id_type=pl.DeviceIdType.MESH)
            rdma_b = pltpu.make_async_remote_copy(
                src_ref=send_b, dst_ref=recv_b.at[s],
                send_sem=rs_ssem_b.at[s], recv_sem=rs_rsem_b.at[s],
                device_id=(left,), device_id_type=pl.DeviceIdType.MESH)
            rdma_a.start()
            rdma_b.start()
            rdma_a.wait()
            rdma_b.wait()
            ca = lax.rem(i + 2 * N_DEV - s - 1, N_DEV)
            cb = lax.rem(i + s + 1, N_DEV)
            sum_a = (p_ref[pl.ds(ca * CH, CH), 0:NH].astype(jnp.float32)
                     + recv_a[s].astype(jnp.float32))
            sum_b = (p_ref[pl.ds(cb * CH, CH), NH:N].astype(jnp.float32)
                     + recv_b[s].astype(jnp.float32))
            if s < N_DEV - 2:
                send_a[...] = sum_a.astype(jnp.bfloat16)
                send_b[...] = sum_b.astype(jnp.bfloat16)
            else:
                acc_a[...] = jnp.maximum(sum_a, 0.0)
                acc_b[...] = jnp.maximum(sum_b, 0.0)

        am_local = jnp.maximum(jnp.max(acc_a[...]), jnp.max(acc_b[...]))
        amax_buf[pl.ds(i, 1), :] = jnp.full((1, 128), am_local, jnp.float32)
        send_descs = []
        for k in range(N_DEV):
            d = pltpu.make_async_remote_copy(
                src_ref=amax_buf.at[pl.ds(i, 1)],
                dst_ref=amax_buf.at[pl.ds(i, 1)],
                send_sem=am_ssem.at[k], recv_sem=am_rsem.at[i],
                device_id=(k,), device_id_type=pl.DeviceIdType.MESH)
            send_descs.append(d)

            @pl.when(i != k)
            def _(d=d):
                d.start()
        for k in range(N_DEV):
            r = pltpu.make_async_remote_copy(
                src_ref=amax_buf.at[pl.ds(k, 1)],
                dst_ref=amax_buf.at[pl.ds(k, 1)],
                send_sem=am_ssem.at[k], recv_sem=am_rsem.at[k],
                device_id=(k,), device_id_type=pl.DeviceIdType.MESH)

            @pl.when(i != k)
            def _(r=r, d=send_descs[k]):
                r.wait_recv()
                d.wait_send()
        amax = jnp.max(amax_buf[...])
        scale = amax / 448.0
        inv_scale = 448.0 / amax

        oa = lax.rem(i + 1, N_DEV)
        ob = lax.rem(i + N_DEV - 1, N_DEV)
        ag_a[pl.ds(oa * CH, CH), :] = (
            (acc_a[...] * inv_scale).astype(jnp.float8_e4m3fn))
        ag_b[pl.ds(ob * CH, CH), :] = (
            (acc_b[...] * inv_scale).astype(jnp.float8_e4m3fn))

        copies_a, copies_b = [], []

        def emit(rows, which):
            copies, base = (copies_a, 0) if which == 0 else (copies_b, 2)
            src = ag_a if which == 0 else ag_b
            col0 = 0 if which == 0 else NH
            j = len(copies)
            slot = base + (j % 2)
            if j >= 2:
                copies[j - 2].wait()
            stage[slot, :, :] = (
                src[rows, :].astype(jnp.float32) * scale
            ).astype(jnp.bfloat16)
            cp = pltpu.make_async_copy(
                stage.at[slot], out_ref.at[rows, col0:col0 + NH],
                copy_sem.at[slot])
            cp.start()
            copies.append(cp)

        emit(pl.ds(oa * CH, CH), 0)
        emit(pl.ds(ob * CH, CH), 1)

        for s in range(N_DEV - 1):
            ca = lax.rem(i + 1 + 2 * N_DEV - s, N_DEV)
            cb = lax.rem(i + N_DEV - 1 + s, N_DEV)
            rdma_a = pltpu.make_async_remote_copy(
                src_ref=ag_a.at[pl.ds(ca * CH, CH)],
                dst_ref=ag_a.at[pl.ds(ca * CH, CH)],
                send_sem=ag_ssem_a.at[s], recv_sem=ag_rsem_a.at[s],
                device_id=(right,), device_id_type=pl.DeviceIdType.MESH)
            rdma_b = pltpu.make_async_remote_copy(
                src_ref=ag_b.at[pl.ds(cb * CH, CH)],
                dst_ref=ag_b.at[pl.ds(cb * CH, CH)],
                send_sem=ag_ssem_b.at[s], recv_sem=ag_rsem_b.at[s],
                device_id=(left,), device_id_type=pl.DeviceIdType.MESH)
            rdma_a.start()
            rdma_b.start()
            rdma_a.wait()
            rdma_b.wait()
            ra = lax.rem(i + 2 * N_DEV - s, N_DEV)
            rb = lax.rem(i + s, N_DEV)
            emit(pl.ds(ra * CH, CH), 0)
            emit(pl.ds(rb * CH, CH), 1)

        for cp in copies_a[-2:]:
            cp.wait()
        for cp in copies_b[-2:]:
            cp.wait()

    nsem = N_DEV - 1
    return pl.pallas_call(
        body,
        out_shape=jax.ShapeDtypeStruct((M, N), jnp.bfloat16),
        in_specs=[pl.BlockSpec(memory_space=pltpu.VMEM)],
        out_specs=pl.BlockSpec(memory_space=pl.ANY),
        scratch_shapes=[
            pltpu.VMEM((nsem, CH, NH), jnp.bfloat16),
            pltpu.VMEM((nsem, CH, NH), jnp.bfloat16),
            pltpu.VMEM((CH, NH), jnp.bfloat16),
            pltpu.VMEM((CH, NH), jnp.bfloat16),
            pltpu.VMEM((CH, NH), jnp.float32),
            pltpu.VMEM((CH, NH), jnp.float32),
            pltpu.VMEM((M, NH), jnp.float8_e4m3fn),
            pltpu.VMEM((M, NH), jnp.float8_e4m3fn),
            pltpu.VMEM((N_DEV, 128), jnp.float32),
            pltpu.VMEM((4, CH, NH), jnp.bfloat16),
            pltpu.SemaphoreType.DMA((nsem,)),
            pltpu.SemaphoreType.DMA((nsem,)),
            pltpu.SemaphoreType.DMA((nsem,)),
            pltpu.SemaphoreType.DMA((nsem,)),
            pltpu.SemaphoreType.DMA((nsem,)),
            pltpu.SemaphoreType.DMA((nsem,)),
            pltpu.SemaphoreType.DMA((nsem,)),
            pltpu.SemaphoreType.DMA((nsem,)),
            pltpu.SemaphoreType.DMA((N_DEV,)),
            pltpu.SemaphoreType.DMA((N_DEV,)),
            pltpu.SemaphoreType.DMA((4,)),
        ],
        compiler_params=pltpu.CompilerParams(
            collective_id=0,
            vmem_limit_bytes=64 * 1024 * 1024,
        ),
    )(partial)


# device time: 169883 ns/iter; 2.3689x vs baseline; 1.1771x over previous
import jax
import jax.numpy as jnp
from jax import lax
from jax.experimental import pallas as pl
from jax.experimental.pallas import tpu as pltpu

N_DEV = 8
M = 4096
N = 2048
NH = N // 2
CH = M // N_DEV
SUB = CH // 2


def kernel(x, w_mat):
    partial = jnp.dot(x, w_mat, preferred_element_type=jnp.float32)
    partial = partial.astype(jnp.bfloat16)

    def body(p_ref, out_ref, recv_a, recv_b, send_a, send_b,
             acc_a, acc_b, ag_a, ag_b, amax_buf, stage,
             rs_ssem_a, rs_rsem_a, rs_ssem_b, rs_rsem_b,
             ag_ssem_a, ag_rsem_a, ag_ssem_b, ag_rsem_b,
             am_ssem, am_rsem, copy_sem):
        i = lax.axis_index("i")
        left = lax.rem(i + N_DEV - 1, N_DEV)
        right = lax.rem(i + 1, N_DEV)

        barrier = pltpu.get_barrier_semaphore()
        for nbr in (left, right):
            pl.semaphore_signal(barrier, inc=1, device_id=(nbr,),
                                device_id_type=pl.DeviceIdType.MESH)
        pl.semaphore_wait(barrier, 2)

        def mk_rs(d, s, sub):
            rows = slice(sub * SUB, (sub + 1) * SUB)
            if d == 0:
                return pltpu.make_async_remote_copy(
                    src_ref=send_a.at[rows], dst_ref=recv_a.at[s, rows],
                    send_sem=rs_ssem_a.at[s, sub],
                    recv_sem=rs_rsem_a.at[s, sub],
                    device_id=(right,), device_id_type=pl.DeviceIdType.MESH)
            return pltpu.make_async_remote_copy(
                src_ref=send_b.at[rows], dst_ref=recv_b.at[s, rows],
                send_sem=rs_ssem_b.at[s, sub],
                recv_sem=rs_rsem_b.at[s, sub],
                device_id=(left,), device_id_type=pl.DeviceIdType.MESH)

        send_a[...] = p_ref[pl.ds(i * CH, CH), 0:NH]
        send_b[...] = p_ref[pl.ds(i * CH, CH), NH:N]
        cur = {}
        for sub in (0, 1):
            for d in (0, 1):
                desc = mk_rs(d, 0, sub)
                desc.start()
                cur[(d, sub)] = desc
        for s in range(N_DEV - 1):
            ca = lax.rem(i + 2 * N_DEV - s - 1, N_DEV)
            cb = lax.rem(i + s + 1, N_DEV)
            for sub in (0, 1):
                rsl = slice(sub * SUB, (sub + 1) * SUB)
                da, db = cur[(0, sub)], cur[(1, sub)]
                da.wait_recv()
                db.wait_recv()
                sum_a = (p_ref[pl.ds(ca * CH + sub * SUB, SUB), 0:NH]
                         .astype(jnp.float32)
                         + recv_a[s, rsl].astype(jnp.float32))
                sum_b = (p_ref[pl.ds(cb * CH + sub * SUB, SUB), NH:N]
                         .astype(jnp.float32)
                         + recv_b[s, rsl].astype(jnp.float32))
                if s < N_DEV - 2:
                    da.wait_send()
                    db.wait_send()
                    send_a[rsl, :] = sum_a.astype(jnp.bfloat16)
                    na = mk_rs(0, s + 1, sub)
                    na.start()
                    cur[(0, sub)] = na
                    send_b[rsl, :] = sum_b.astype(jnp.bfloat16)
                    nb = mk_rs(1, s + 1, sub)
                    nb.start()
                    cur[(1, sub)] = nb
                else:
                    acc_a[rsl, :] = jnp.maximum(sum_a, 0.0)
                    acc_b[rsl, :] = jnp.maximum(sum_b, 0.0)
                    da.wait_send()
                    db.wait_send()

        am_local = jnp.maximum(jnp.max(acc_a[...]), jnp.max(acc_b[...]))
        amax_buf[pl.ds(i, 1), :] = jnp.full((1, 128), am_local, jnp.float32)
        send_descs = []
        for k in range(N_DEV):
            d = pltpu.make_async_remote_copy(
                src_ref=amax_buf.at[pl.ds(i, 1)],
                dst_ref=amax_buf.at[pl.ds(i, 1)],
                send_sem=am_ssem.at[k], recv_sem=am_rsem.at[i],
                device_id=(k,), device_id_type=pl.DeviceIdType.MESH)
            send_descs.append(d)

            @pl.when(i != k)
            def _(d=d):
                d.start()
        for k in range(N_DEV):
            r = pltpu.make_async_remote_copy(
                src_ref=amax_buf.at[pl.ds(k, 1)],
                dst_ref=amax_buf.at[pl.ds(k, 1)],
                send_sem=am_ssem.at[k], recv_sem=am_rsem.at[k],
                device_id=(k,), device_id_type=pl.DeviceIdType.MESH)

            @pl.when(i != k)
            def _(r=r, d=send_descs[k]):
                r.wait_recv()
                d.wait_send()
        amax = jnp.max(amax_buf[...])
        scale = amax / 448.0
        inv_scale = 448.0 / amax

        oa = lax.rem(i + 1, N_DEV)
        ob = lax.rem(i + N_DEV - 1, N_DEV)
        ag_a[pl.ds(oa * CH, CH), :] = (
            (acc_a[...] * inv_scale).astype(jnp.float8_e4m3fn))
        ag_b[pl.ds(ob * CH, CH), :] = (
            (acc_b[...] * inv_scale).astype(jnp.float8_e4m3fn))

        copies_a, copies_b = [], []

        def emit(rows, which):
            copies, base = (copies_a, 0) if which == 0 else (copies_b, 2)
            src = ag_a if which == 0 else ag_b
            col0 = 0 if which == 0 else NH
            j = len(copies)
            slot = base + (j % 2)
            if j >= 2:
                copies[j - 2].wait()
            stage[slot, :, :] = (
                src[rows, :].astype(jnp.float32) * scale
            ).astype(jnp.bfloat16)
            cp = pltpu.make_async_copy(
                stage.at[slot], out_ref.at[rows, col0:col0 + NH],
                copy_sem.at[slot])
            cp.start()
            copies.append(cp)

        emit(pl.ds(oa * CH, CH), 0)
        emit(pl.ds(ob * CH, CH), 1)

        def mk_ag(d, s, sub, chunk):
            rows = pl.ds(chunk * CH + sub * SUB, SUB)
            if d == 0:
                return pltpu.make_async_remote_copy(
                    src_ref=ag_a.at[rows], dst_ref=ag_a.at[rows],
                    send_sem=ag_ssem_a.at[s, sub],
                    recv_sem=ag_rsem_a.at[s, sub],
                    device_id=(right,), device_id_type=pl.DeviceIdType.MESH)
            return pltpu.make_async_remote_copy(
                src_ref=ag_b.at[rows], dst_ref=ag_b.at[rows],
                send_sem=ag_ssem_b.at[s, sub],
                recv_sem=ag_rsem_b.at[s, sub],
                device_id=(left,), device_id_type=pl.DeviceIdType.MESH)

        curg = {}
        for sub in (0, 1):
            for d, own in ((0, oa), (1, ob)):
                desc = mk_ag(d, 0, sub, own)
                desc.start()
                curg[(d, sub)] = desc
        for s in range(N_DEV - 1):
            ra = lax.rem(i + 2 * N_DEV - s, N_DEV)
            rb = lax.rem(i + s, N_DEV)
            prev = []
            for sub in (0, 1):
                da, db = curg[(0, sub)], curg[(1, sub)]
                da.wait_recv()
                db.wait_recv()
                prev += [da, db]
                if s < N_DEV - 2:
                    na = mk_ag(0, s + 1, sub, ra)
                    na.start()
                    curg[(0, sub)] = na
                    nb = mk_ag(1, s + 1, sub, rb)
                    nb.start()
                    curg[(1, sub)] = nb
            emit(pl.ds(ra * CH, CH), 0)
            emit(pl.ds(rb * CH, CH), 1)
            for dsc in prev:
                dsc.wait_send()

        for cp in copies_a[-2:]:
            cp.wait()
        for cp in copies_b[-2:]:
            cp.wait()

    nsem = N_DEV - 1
    return pl.pallas_call(
        body,
        out_shape=jax.ShapeDtypeStruct((M, N), jnp.bfloat16),
        in_specs=[pl.BlockSpec(memory_space=pltpu.VMEM)],
        out_specs=pl.BlockSpec(memory_space=pl.ANY),
        scratch_shapes=[
            pltpu.VMEM((nsem, CH, NH), jnp.bfloat16),
            pltpu.VMEM((nsem, CH, NH), jnp.bfloat16),
            pltpu.VMEM((CH, NH), jnp.bfloat16),
            pltpu.VMEM((CH, NH), jnp.bfloat16),
            pltpu.VMEM((CH, NH), jnp.float32),
            pltpu.VMEM((CH, NH), jnp.float32),
            pltpu.VMEM((M, NH), jnp.float8_e4m3fn),
            pltpu.VMEM((M, NH), jnp.float8_e4m3fn),
            pltpu.VMEM((N_DEV, 128), jnp.float32),
            pltpu.VMEM((4, CH, NH), jnp.bfloat16),
            pltpu.SemaphoreType.DMA((nsem, 2)),
            pltpu.SemaphoreType.DMA((nsem, 2)),
            pltpu.SemaphoreType.DMA((nsem, 2)),
            pltpu.SemaphoreType.DMA((nsem, 2)),
            pltpu.SemaphoreType.DMA((nsem, 2)),
            pltpu.SemaphoreType.DMA((nsem, 2)),
            pltpu.SemaphoreType.DMA((nsem, 2)),
            pltpu.SemaphoreType.DMA((nsem, 2)),
            pltpu.SemaphoreType.DMA((N_DEV,)),
            pltpu.SemaphoreType.DMA((N_DEV,)),
            pltpu.SemaphoreType.DMA((4,)),
        ],
        compiler_params=pltpu.CompilerParams(
            collective_id=0,
            vmem_limit_bytes=64 * 1024 * 1024,
        ),
    )(partial)


# device time: 153537 ns/iter; 2.6211x vs baseline; 1.1065x over previous
import jax
import jax.numpy as jnp
from jax import lax
from jax.experimental import pallas as pl
from jax.experimental.pallas import tpu as pltpu

N_DEV = 8
M = 4096
N = 2048
NH = N // 2
CH = M // N_DEV
SUB = CH // 2


def kernel(x, w_mat):
    def body(x_ref, w_ref, out_ref, recv_a, recv_b, send_a, send_b,
             acc_a, acc_b, ag_a, ag_b, amax_buf, stage,
             rs_ssem_a, rs_rsem_a, rs_ssem_b, rs_rsem_b,
             ag_ssem_a, ag_rsem_a, ag_ssem_b, ag_rsem_b,
             am_ssem, am_rsem, copy_sem):
        i = lax.axis_index("i")
        left = lax.rem(i + N_DEV - 1, N_DEV)
        right = lax.rem(i + 1, N_DEV)

        barrier = pltpu.get_barrier_semaphore()
        for nbr in (left, right):
            pl.semaphore_signal(barrier, inc=1, device_id=(nbr,),
                                device_id_type=pl.DeviceIdType.MESH)
        pl.semaphore_wait(barrier, 2)

        def mk_rs(d, s, sub):
            rows = slice(sub * SUB, (sub + 1) * SUB)
            if d == 0:
                return pltpu.make_async_remote_copy(
                    src_ref=send_a.at[rows], dst_ref=recv_a.at[s, rows],
                    send_sem=rs_ssem_a.at[s, sub],
                    recv_sem=rs_rsem_a.at[s, sub],
                    device_id=(right,), device_id_type=pl.DeviceIdType.MESH)
            return pltpu.make_async_remote_copy(
                src_ref=send_b.at[rows], dst_ref=recv_b.at[s, rows],
                send_sem=rs_ssem_b.at[s, sub],
                recv_sem=rs_rsem_b.at[s, sub],
                device_id=(left,), device_id_type=pl.DeviceIdType.MESH)

        p_own = jnp.dot(x_ref[pl.ds(i * CH, CH), :], w_ref[...],
                        preferred_element_type=jnp.float32)
        send_a[...] = p_own[:, 0:NH].astype(jnp.bfloat16)
        send_b[...] = p_own[:, NH:N].astype(jnp.bfloat16)
        cur = {}
        for sub in (0, 1):
            for d in (0, 1):
                desc = mk_rs(d, 0, sub)
                desc.start()
                cur[(d, sub)] = desc
        for s in range(N_DEV - 1):
            ca = lax.rem(i + 2 * N_DEV - s - 1, N_DEV)
            cb = lax.rem(i + s + 1, N_DEV)
            for sub in (0, 1):
                rsl = slice(sub * SUB, (sub + 1) * SUB)
                da, db = cur[(0, sub)], cur[(1, sub)]
                pa = jnp.dot(x_ref[pl.ds(ca * CH + sub * SUB, SUB), :],
                             w_ref[:, 0:NH],
                             preferred_element_type=jnp.float32)
                pb = jnp.dot(x_ref[pl.ds(cb * CH + sub * SUB, SUB), :],
                             w_ref[:, NH:N],
                             preferred_element_type=jnp.float32)
                da.wait_recv()
                db.wait_recv()
                sum_a = pa + recv_a[s, rsl].astype(jnp.float32)
                sum_b = pb + recv_b[s, rsl].astype(jnp.float32)
                if s < N_DEV - 2:
                    da.wait_send()
                    db.wait_send()
                    send_a[rsl, :] = sum_a.astype(jnp.bfloat16)
                    na = mk_rs(0, s + 1, sub)
                    na.start()
                    cur[(0, sub)] = na
                    send_b[rsl, :] = sum_b.astype(jnp.bfloat16)
                    nb = mk_rs(1, s + 1, sub)
                    nb.start()
                    cur[(1, sub)] = nb
                else:
                    acc_a[rsl, :] = jnp.maximum(sum_a, 0.0)
                    acc_b[rsl, :] = jnp.maximum(sum_b, 0.0)
                    da.wait_send()
                    db.wait_send()

        am_local = jnp.maximum(jnp.max(acc_a[...]), jnp.max(acc_b[...]))
        amax_buf[pl.ds(i, 1), :] = jnp.full((1, 128), am_local, jnp.float32)
        send_descs = []
        for k in range(N_DEV):
            d = pltpu.make_async_remote_copy(
                src_ref=amax_buf.at[pl.ds(i, 1)],
                dst_ref=amax_buf.at[pl.ds(i, 1)],
                send_sem=am_ssem.at[k], recv_sem=am_rsem.at[i],
                device_id=(k,), device_id_type=pl.DeviceIdType.MESH)
            send_descs.append(d)

            @pl.when(i != k)
            def _(d=d):
                d.start()
        for k in range(N_DEV):
            r = pltpu.make_async_remote_copy(
                src_ref=amax_buf.at[pl.ds(k, 1)],
                dst_ref=amax_buf.at[pl.ds(k, 1)],
                send_sem=am_ssem.at[k], recv_sem=am_rsem.at[k],
                device_id=(k,), device_id_type=pl.DeviceIdType.MESH)

            @pl.when(i != k)
            def _(r=r, d=send_descs[k]):
                r.wait_recv()
                d.wait_send()
        amax = jnp.max(amax_buf[...])
        scale = amax / 448.0
        inv_scale = 448.0 / amax

        oa = lax.rem(i + 1, N_DEV)
        ob = lax.rem(i + N_DEV - 1, N_DEV)
        ag_a[pl.ds(oa * CH, CH), :] = (
            (acc_a[...] * inv_scale).astype(jnp.float8_e4m3fn))
        ag_b[pl.ds(ob * CH, CH), :] = (
            (acc_b[...] * inv_scale).astype(jnp.float8_e4m3fn))

        copies_a, copies_b = [], []

        def emit(rows, which):
            copies, base = (copies_a, 0) if which == 0 else (copies_b, 2)
            src = ag_a if which == 0 else ag_b
            col0 = 0 if which == 0 else NH
            j = len(copies)
            slot = base + (j % 2)
            if j >= 2:
                copies[j - 2].wait()
            stage[slot, :, :] = (
                src[rows, :].astype(jnp.float32) * scale
            ).astype(jnp.bfloat16)
            cp = pltpu.make_async_copy(
                stage.at[slot], out_ref.at[rows, col0:col0 + NH],
                copy_sem.at[slot])
            cp.start()
            copies.append(cp)

        emit(pl.ds(oa * CH, CH), 0)
        emit(pl.ds(ob * CH, CH), 1)

        def mk_ag(d, s, sub, chunk):
            rows = pl.ds(chunk * CH + sub * SUB, SUB)
            if d == 0:
                return pltpu.make_async_remote_copy(
                    src_ref=ag_a.at[rows], dst_ref=ag_a.at[rows],
                    send_sem=ag_ssem_a.at[s, sub],
                    recv_sem=ag_rsem_a.at[s, sub],
                    device_id=(right,), device_id_type=pl.DeviceIdType.MESH)
            return pltpu.make_async_remote_copy(
                src_ref=ag_b.at[rows], dst_ref=ag_b.at[rows],
                send_sem=ag_ssem_b.at[s, sub],
                recv_sem=ag_rsem_b.at[s, sub],
                device_id=(left,), device_id_type=pl.DeviceIdType.MESH)

        curg = {}
        for sub in (0, 1):
            for d, own in ((0, oa), (1, ob)):
                desc = mk_ag(d, 0, sub, own)
                desc.start()
                curg[(d, sub)] = desc
        for s in range(N_DEV - 1):
            ra = lax.rem(i + 2 * N_DEV - s, N_DEV)
            rb = lax.rem(i + s, N_DEV)
            prev = []
            for sub in (0, 1):
                da, db = curg[(0, sub)], curg[(1, sub)]
                da.wait_recv()
                db.wait_recv()
                prev += [da, db]
                if s < N_DEV - 2:
                    na = mk_ag(0, s + 1, sub, ra)
                    na.start()
                    curg[(0, sub)] = na
                    nb = mk_ag(1, s + 1, sub, rb)
                    nb.start()
                    curg[(1, sub)] = nb
            emit(pl.ds(ra * CH, CH), 0)
            emit(pl.ds(rb * CH, CH), 1)
            for dsc in prev:
                dsc.wait_send()

        for cp in copies_a[-2:]:
            cp.wait()
        for cp in copies_b[-2:]:
            cp.wait()

    nsem = N_DEV - 1
    return pl.pallas_call(
        body,
        out_shape=jax.ShapeDtypeStruct((M, N), jnp.bfloat16),
        in_specs=[pl.BlockSpec(memory_space=pltpu.VMEM),
                  pl.BlockSpec(memory_space=pltpu.VMEM)],
        out_specs=pl.BlockSpec(memory_space=pl.ANY),
        scratch_shapes=[
            pltpu.VMEM((nsem, CH, NH), jnp.bfloat16),
            pltpu.VMEM((nsem, CH, NH), jnp.bfloat16),
            pltpu.VMEM((CH, NH), jnp.bfloat16),
            pltpu.VMEM((CH, NH), jnp.bfloat16),
            pltpu.VMEM((CH, NH), jnp.float32),
            pltpu.VMEM((CH, NH), jnp.float32),
            pltpu.VMEM((M, NH), jnp.float8_e4m3fn),
            pltpu.VMEM((M, NH), jnp.float8_e4m3fn),
            pltpu.VMEM((N_DEV, 128), jnp.float32),
            pltpu.VMEM((4, CH, NH), jnp.bfloat16),
            pltpu.SemaphoreType.DMA((nsem, 2)),
            pltpu.SemaphoreType.DMA((nsem, 2)),
            pltpu.SemaphoreType.DMA((nsem, 2)),
            pltpu.SemaphoreType.DMA((nsem, 2)),
            pltpu.SemaphoreType.DMA((nsem, 2)),
            pltpu.SemaphoreType.DMA((nsem, 2)),
            pltpu.SemaphoreType.DMA((nsem, 2)),
            pltpu.SemaphoreType.DMA((nsem, 2)),
            pltpu.SemaphoreType.DMA((N_DEV,)),
            pltpu.SemaphoreType.DMA((N_DEV,)),
            pltpu.SemaphoreType.DMA((4,)),
        ],
        compiler_params=pltpu.CompilerParams(
            collective_id=0,
            vmem_limit_bytes=64 * 1024 * 1024,
        ),
    )(x, w_mat)


# device time: 152901 ns/iter; 2.6320x vs baseline; 1.0042x over previous
import jax
import jax.numpy as jnp
from jax import lax
from jax.experimental import pallas as pl
from jax.experimental.pallas import tpu as pltpu

N_DEV = 8
M = 4096
N = 2048
NH = N // 2
CH = M // N_DEV
SUB = CH // 2


def kernel(x, w_mat):
    def body(x_ref, w_ref, out_ref, recv_a, recv_b, send_a, send_b,
             acc_a, acc_b, ag_a, ag_b, amax_buf, stage,
             rs_ssem_a, rs_rsem_a, rs_ssem_b, rs_rsem_b,
             ag_ssem_a, ag_rsem_a, ag_ssem_b, ag_rsem_b,
             am_ssem, am_rsem, copy_sem):
        i = lax.axis_index("i")

        def ring2dev(q):
            q = lax.rem(q + 2 * N_DEV, N_DEV)
            return jnp.where(q < 4, q, 11 - q)

        r = jnp.where(i < 4, i, 11 - i)
        left = ring2dev(r - 1)
        right = ring2dev(r + 1)

        barrier = pltpu.get_barrier_semaphore()
        for nbr in (left, right):
            pl.semaphore_signal(barrier, inc=1, device_id=(nbr,),
                                device_id_type=pl.DeviceIdType.MESH)
        pl.semaphore_wait(barrier, 2)

        def mk_rs(d, s, sub):
            rows = slice(sub * SUB, (sub + 1) * SUB)
            if d == 0:
                return pltpu.make_async_remote_copy(
                    src_ref=send_a.at[rows], dst_ref=recv_a.at[s, rows],
                    send_sem=rs_ssem_a.at[s, sub],
                    recv_sem=rs_rsem_a.at[s, sub],
                    device_id=(right,), device_id_type=pl.DeviceIdType.MESH)
            return pltpu.make_async_remote_copy(
                src_ref=send_b.at[rows], dst_ref=recv_b.at[s, rows],
                send_sem=rs_ssem_b.at[s, sub],
                recv_sem=rs_rsem_b.at[s, sub],
                device_id=(left,), device_id_type=pl.DeviceIdType.MESH)

        p_own = jnp.dot(x_ref[pl.ds(i * CH, CH), :], w_ref[...],
                        preferred_element_type=jnp.float32)
        send_a[...] = p_own[:, 0:NH].astype(jnp.bfloat16)
        send_b[...] = p_own[:, NH:N].astype(jnp.bfloat16)
        cur = {}
        for sub in (0, 1):
            for d in (0, 1):
                desc = mk_rs(d, 0, sub)
                desc.start()
                cur[(d, sub)] = desc
        for s in range(N_DEV - 1):
            ca = ring2dev(r - s - 1)
            cb = ring2dev(r + s + 1)
            for sub in (0, 1):
                rsl = slice(sub * SUB, (sub + 1) * SUB)
                da, db = cur[(0, sub)], cur[(1, sub)]
                pa = jnp.dot(x_ref[pl.ds(ca * CH + sub * SUB, SUB), :],
                             w_ref[:, 0:NH],
                             preferred_element_type=jnp.float32)
                pb = jnp.dot(x_ref[pl.ds(cb * CH + sub * SUB, SUB), :],
                             w_ref[:, NH:N],
                             preferred_element_type=jnp.float32)
                da.wait_recv()
                db.wait_recv()
                sum_a = pa + recv_a[s, rsl].astype(jnp.float32)
                sum_b = pb + recv_b[s, rsl].astype(jnp.float32)
                if s < N_DEV - 2:
                    da.wait_send()
                    db.wait_send()
                    send_a[rsl, :] = sum_a.astype(jnp.bfloat16)
                    na = mk_rs(0, s + 1, sub)
                    na.start()
                    cur[(0, sub)] = na
                    send_b[rsl, :] = sum_b.astype(jnp.bfloat16)
                    nb = mk_rs(1, s + 1, sub)
                    nb.start()
                    cur[(1, sub)] = nb
                else:
                    acc_a[rsl, :] = jnp.maximum(sum_a, 0.0)
                    acc_b[rsl, :] = jnp.maximum(sum_b, 0.0)
                    da.wait_send()
                    db.wait_send()

        am_local = jnp.maximum(jnp.max(acc_a[...]), jnp.max(acc_b[...]))
        amax_buf[pl.ds(i, 1), :] = jnp.full((1, 128), am_local, jnp.float32)
        send_descs = []
        for k in range(N_DEV):
            d = pltpu.make_async_remote_copy(
                src_ref=amax_buf.at[pl.ds(i, 1)],
                dst_ref=amax_buf.at[pl.ds(i, 1)],
                send_sem=am_ssem.at[k], recv_sem=am_rsem.at[i],
                device_id=(k,), device_id_type=pl.DeviceIdType.MESH)
            send_descs.append(d)

            @pl.when(i != k)
            def _(d=d):
                d.start()
        for k in range(N_DEV):
            rcv = pltpu.make_async_remote_copy(
                src_ref=amax_buf.at[pl.ds(k, 1)],
                dst_ref=amax_buf.at[pl.ds(k, 1)],
                send_sem=am_ssem.at[k], recv_sem=am_rsem.at[k],
                device_id=(k,), device_id_type=pl.DeviceIdType.MESH)

            @pl.when(i != k)
            def _(rcv=rcv, d=send_descs[k]):
                rcv.wait_recv()
                d.wait_send()
        amax = jnp.max(amax_buf[...])
        scale = amax / 448.0
        inv_scale = 448.0 / amax

        oa = right
        ob = left
        ag_a[pl.ds(oa * CH, CH), :] = (
            (acc_a[...] * inv_scale).astype(jnp.float8_e4m3fn))
        ag_b[pl.ds(ob * CH, CH), :] = (
            (acc_b[...] * inv_scale).astype(jnp.float8_e4m3fn))

        copies_a, copies_b = [], []

        def emit(rows, which):
            copies, base = (copies_a, 0) if which == 0 else (copies_b, 2)
            src = ag_a if which == 0 else ag_b
            col0 = 0 if which == 0 else NH
            j = len(copies)
            slot = base + (j % 2)
            if j >= 2:
                copies[j - 2].wait()
            stage[slot, :, :] = (
                src[rows, :].astype(jnp.float32) * scale
            ).astype(jnp.bfloat16)
            cp = pltpu.make_async_copy(
                stage.at[slot], out_ref.at[rows, col0:col0 + NH],
                copy_sem.at[slot])
            cp.start()
            copies.append(cp)

        emit(pl.ds(oa * CH, CH), 0)
        emit(pl.ds(ob * CH, CH), 1)

        def mk_ag(d, s, sub, chunk):
            rows = pl.ds(chunk * CH + sub * SUB, SUB)
            if d == 0:
                return pltpu.make_async_remote_copy(
                    src_ref=ag_a.at[rows], dst_ref=ag_a.at[rows],
                    send_sem=ag_ssem_a.at[s, sub],
                    recv_sem=ag_rsem_a.at[s, sub],
                    device_id=(right,), device_id_type=pl.DeviceIdType.MESH)
            return pltpu.make_async_remote_copy(
                src_ref=ag_b.at[rows], dst_ref=ag_b.at[rows],
                send_sem=ag_ssem_b.at[s, sub],
                recv_sem=ag_rsem_b.at[s, sub],
                device_id=(left,), device_id_type=pl.DeviceIdType.MESH)

        curg = {}
        for sub in (0, 1):
            for d, own in ((0, oa), (1, ob)):
                desc = mk_ag(d, 0, sub, own)
                desc.start()
                curg[(d, sub)] = desc
        for s in range(N_DEV - 1):
            ra = ring2dev(r - s)
            rb = ring2dev(r + s)
            prev = []
            for sub in (0, 1):
                da, db = curg[(0, sub)], curg[(1, sub)]
                da.wait_recv()
                db.wait_recv()
                prev += [da, db]
                if s < N_DEV - 2:
                    na = mk_ag(0, s + 1, sub, ra)
                    na.start()
                    curg[(0, sub)] = na
                    nb = mk_ag(1, s + 1, sub, rb)
                    nb.start()
                    curg[(1, sub)] = nb
            emit(pl.ds(ra * CH, CH), 0)
            emit(pl.ds(rb * CH, CH), 1)
            for dsc in prev:
                dsc.wait_send()

        for cp in copies_a[-2:]:
            cp.wait()
        for cp in copies_b[-2:]:
            cp.wait()

    nsem = N_DEV - 1
    return pl.pallas_call(
        body,
        out_shape=jax.ShapeDtypeStruct((M, N), jnp.bfloat16),
        in_specs=[pl.BlockSpec(memory_space=pltpu.VMEM),
                  pl.BlockSpec(memory_space=pltpu.VMEM)],
        out_specs=pl.BlockSpec(memory_space=pl.ANY),
        scratch_shapes=[
            pltpu.VMEM((nsem, CH, NH), jnp.bfloat16),
            pltpu.VMEM((nsem, CH, NH), jnp.bfloat16),
            pltpu.VMEM((CH, NH), jnp.bfloat16),
            pltpu.VMEM((CH, NH), jnp.bfloat16),
            pltpu.VMEM((CH, NH), jnp.float32),
            pltpu.VMEM((CH, NH), jnp.float32),
            pltpu.VMEM((M, NH), jnp.float8_e4m3fn),
            pltpu.VMEM((M, NH), jnp.float8_e4m3fn),
            pltpu.VMEM((N_DEV, 128), jnp.float32),
            pltpu.VMEM((4, CH, NH), jnp.bfloat16),
            pltpu.SemaphoreType.DMA((nsem, 2)),
            pltpu.SemaphoreType.DMA((nsem, 2)),
            pltpu.SemaphoreType.DMA((nsem, 2)),
            pltpu.SemaphoreType.DMA((nsem, 2)),
            pltpu.SemaphoreType.DMA((nsem, 2)),
            pltpu.SemaphoreType.DMA((nsem, 2)),
            pltpu.SemaphoreType.DMA((nsem, 2)),
            pltpu.SemaphoreType.DMA((nsem, 2)),
            pltpu.SemaphoreType.DMA((N_DEV,)),
            pltpu.SemaphoreType.DMA((N_DEV,)),
            pltpu.SemaphoreType.DMA((4,)),
        ],
        compiler_params=pltpu.CompilerParams(
            collective_id=0,
            vmem_limit_bytes=64 * 1024 * 1024,
        ),
    )(x, w_mat)
